# Initial kernel scaffold; baseline (speedup 1.0000x reference)
import jax
import jax.numpy as jnp
from jax.experimental import pallas as pl


def kernel(pred_logits, pred_boxes, tgt_labels, tgt_boxes):
    raise NotImplementedError("write your pallas kernel here")


# speedup vs baseline: 16.8017x; 16.8017x over previous
import jax
import jax.numpy as jnp
from jax.experimental import pallas as pl

_B, _Q, _C, _T, _K = 8, 900, 91, 300, 6
_COST_CLASS, _COST_BBOX, _COST_GIOU = 1.0, 5.0, 2.0
_EPS = 1e-06
_ALPHA = 0.25


def _matcher_kernel(logits_ref, pbT_ref, labels_ref, tb_ref, idxq_ref, idxt_ref):
    logits = jnp.nan_to_num(logits_ref[0], nan=0.0)
    prob = jax.nn.sigmoid(logits)
    pos = _ALPHA * ((1.0 - prob) * (1.0 - prob)) * -jnp.log(prob + 1e-08)
    neg = (1.0 - _ALPHA) * (prob * prob) * -jnp.log(1.0 - prob + 1e-08)
    d = pos - neg
    labels = labels_ref[0]
    onehot = (labels == jax.lax.broadcasted_iota(jnp.int32, (_T, _C), 1)
              ).astype(jnp.float32)
    c_cls = jax.lax.dot_general(
        onehot, d, (((1,), (1,)), ((), ())),
        preferred_element_type=jnp.float32,
        precision=jax.lax.Precision.HIGHEST)

    pbT = jax.nn.sigmoid(pbT_ref[0])
    qcx, qcy = pbT[0:1, :], pbT[1:2, :]
    qw, qh = pbT[2:3, :], pbT[3:4, :]
    tb = jnp.clip(tb_ref[0], 0.0, 1.0)
    tx1, ty1 = tb[:, 0:1], tb[:, 1:2]
    tx2, ty2 = tb[:, 2:3], tb[:, 3:4]
    tw = jnp.maximum(tx2 - tx1, 1e-05)
    th = jnp.maximum(ty2 - ty1, 1e-05)
    tcx = (tx1 + tx2) * 0.5
    tcy = (ty1 + ty2) * 0.5

    c_l1 = (jnp.abs(qcx - tcx) + jnp.abs(qcy - tcy)
            + jnp.abs(qw - tw) + jnp.abs(qh - th))

    qx1 = jnp.clip(qcx - 0.5 * qw, 0.0, 1.0)
    qy1 = jnp.clip(qcy - 0.5 * qh, 0.0, 1.0)
    qx2 = jnp.clip(qcx + 0.5 * qw, 0.0, 1.0)
    qy2 = jnp.clip(qcy + 0.5 * qh, 0.0, 1.0)
    lt_x = jnp.maximum(qx1, tx1)
    lt_y = jnp.maximum(qy1, ty1)
    rb_x = jnp.minimum(qx2, tx2)
    rb_y = jnp.minimum(qy2, ty2)
    inter = jnp.maximum(rb_x - lt_x, 0.0) * jnp.maximum(rb_y - lt_y, 0.0)
    area_q = jnp.maximum(qx2 - qx1, 0.0) * jnp.maximum(qy2 - qy1, 0.0)
    area_t = jnp.maximum(tx2 - tx1, 0.0) * jnp.maximum(ty2 - ty1, 0.0)
    union = jnp.maximum(area_q + area_t - inter, _EPS)
    iou = inter / union
    en_x = jnp.maximum(qx2, tx2) - jnp.minimum(qx1, tx1)
    en_y = jnp.maximum(qy2, ty2) - jnp.minimum(qy1, ty1)
    area_c = jnp.maximum(jnp.maximum(en_x, 0.0) * jnp.maximum(en_y, 0.0), _EPS)
    giou = jnp.clip(iou - (area_c - union) / area_c, -1.0, 1.0)
    c_iou = 1.0 - giou

    cost = _COST_CLASS * c_cls + _COST_BBOX * c_l1 + _COST_GIOU * c_iou

    iota_q = jax.lax.broadcasted_iota(jnp.int32, (_T, _Q), 1)
    for j in range(_K):
        m = jnp.min(cost, axis=1, keepdims=True)
        idx = jnp.min(jnp.where(cost == m, iota_q, _Q),
                      axis=1, keepdims=True)
        idxq_ref[0, :, pl.ds(j, 1)] = idx
        cost = jnp.where(iota_q == idx, jnp.inf, cost)
    idxt_ref[0] = jax.lax.broadcasted_iota(jnp.int32, (_T, _K), 0)


def kernel(pred_logits, pred_boxes, tgt_labels, tgt_boxes):
    pbT = pred_boxes.astype(jnp.float32).transpose(0, 2, 1)
    labels3 = tgt_labels.reshape(_B, _T, 1)
    out_q, out_t = pl.pallas_call(
        _matcher_kernel,
        grid=(_B,),
        in_specs=[
            pl.BlockSpec((1, _Q, _C), lambda b: (b, 0, 0)),
            pl.BlockSpec((1, 4, _Q), lambda b: (b, 0, 0)),
            pl.BlockSpec((1, _T, 1), lambda b: (b, 0, 0)),
            pl.BlockSpec((1, _T, 4), lambda b: (b, 0, 0)),
        ],
        out_specs=[
            pl.BlockSpec((1, _T, _K), lambda b: (b, 0, 0)),
            pl.BlockSpec((1, _T, _K), lambda b: (b, 0, 0)),
        ],
        out_shape=[
            jax.ShapeDtypeStruct((_B, _T, _K), jnp.int32),
            jax.ShapeDtypeStruct((_B, _T, _K), jnp.int32),
        ],
    )(pred_logits.astype(jnp.float32), pbT, labels3, tgt_boxes)
    idx_q = out_q.transpose(0, 2, 1).reshape(_B, _K * _T)
    idx_t = out_t.reshape(_B, _K * _T)
    return idx_q, idx_t
